# baseline (speedup 1.0000x reference)
import functools

import jax
import jax.numpy as jnp
from jax.experimental import pallas as pl
from jax.experimental.pallas import tpu as pltpu

SCALE = 0.08838834764831845
NUM_HEADS = 16
HEAD_SIZE = 128
NUM_PROMPTS = 2
PROMPT_LEN = 2048
NUM_GEN = 16
BLOCK_SIZE = 16
X = 8
NUM_BLOCKS = 512
MAX_CTX = 1024
NEG_INF = -1e30

QBLK = 512
KBLK = 512


def _flash_body(q_ref, k_ref, v_ref, o_ref):
    qi = pl.program_id(2)
    q = q_ref[0, 0, :, :] * SCALE

    def body(kb, carry):
        m, l, acc = carry
        k = k_ref[0, 0, pl.ds(kb * KBLK, KBLK), :]
        v = v_ref[0, 0, pl.ds(kb * KBLK, KBLK), :]
        s = jax.lax.dot_general(q, k, (((1,), (1,)), ((), ())),
                                preferred_element_type=jnp.float32)
        qpos = qi * QBLK + jax.lax.broadcasted_iota(jnp.int32, (QBLK, KBLK), 0)
        kpos = kb * KBLK + jax.lax.broadcasted_iota(jnp.int32, (QBLK, KBLK), 1)
        s = jnp.where(qpos >= kpos, s, NEG_INF)
        m_new = jnp.maximum(m, jnp.max(s, axis=1, keepdims=True))
        p = jnp.exp(s - m_new)
        alpha = jnp.exp(m - m_new)
        l_new = l * alpha + jnp.sum(p, axis=1, keepdims=True)
        acc_new = acc * alpha + jax.lax.dot_general(
            p, v, (((1,), (0,)), ((), ())), preferred_element_type=jnp.float32)
        return m_new, l_new, acc_new

    m0 = jnp.full((QBLK, 1), NEG_INF, jnp.float32)
    l0 = jnp.zeros((QBLK, 1), jnp.float32)
    a0 = jnp.zeros((QBLK, HEAD_SIZE), jnp.float32)
    m, l, acc = jax.lax.fori_loop(0, qi + 1, body, (m0, l0, a0))
    o_ref[0, 0, :, :] = acc / l


def _prompt_attention(qp, kp, vp):
    grid = (NUM_PROMPTS, NUM_HEADS, PROMPT_LEN // QBLK)
    return pl.pallas_call(
        _flash_body,
        grid=grid,
        in_specs=[
            pl.BlockSpec((1, 1, QBLK, HEAD_SIZE), lambda p, h, q: (p, h, q, 0)),
            pl.BlockSpec((1, 1, PROMPT_LEN, HEAD_SIZE), lambda p, h, q: (p, h, 0, 0)),
            pl.BlockSpec((1, 1, PROMPT_LEN, HEAD_SIZE), lambda p, h, q: (p, h, 0, 0)),
        ],
        out_specs=pl.BlockSpec((1, 1, QBLK, HEAD_SIZE), lambda p, h, q: (p, h, q, 0)),
        out_shape=jax.ShapeDtypeStruct(
            (NUM_PROMPTS, NUM_HEADS, PROMPT_LEN, HEAD_SIZE), jnp.float32),
        compiler_params=pltpu.CompilerParams(
            dimension_semantics=("parallel", "parallel", "arbitrary")),
    )(qp, kp, vp)


def _patch_body(tok_ref, *refs):
    kr_refs = refs[:BLOCK_SIZE]
    vr_refs = refs[BLOCK_SIZE:2 * BLOCK_SIZE]
    kp_ref, vp_ref = refs[2 * BLOCK_SIZE:]
    for off in range(BLOCK_SIZE):
        kp_ref[0, :, off, :] = kr_refs[off][0]
        vp_ref[0, :, off, :] = vr_refs[off][0]


def _build_patches(k3, v3, slot_mapping):
    tok_map = jnp.zeros((NUM_BLOCKS * BLOCK_SIZE,), jnp.int32)
    tok_map = tok_map.at[slot_mapping].set(
        jnp.arange(k3.shape[0], dtype=jnp.int32))

    def _row_map(off):
        return lambda b, t: (t[b * BLOCK_SIZE + off], 0, 0)

    grid_spec = pltpu.PrefetchScalarGridSpec(
        num_scalar_prefetch=1,
        grid=(NUM_BLOCKS,),
        in_specs=[
            pl.BlockSpec((1, NUM_HEADS, HEAD_SIZE), _row_map(off))
            for off in range(BLOCK_SIZE)
        ] + [
            pl.BlockSpec((1, NUM_HEADS, HEAD_SIZE), _row_map(off))
            for off in range(BLOCK_SIZE)
        ],
        out_specs=[
            pl.BlockSpec((1, NUM_HEADS, BLOCK_SIZE, HEAD_SIZE),
                         lambda b, t: (b, 0, 0, 0)),
            pl.BlockSpec((1, NUM_HEADS, BLOCK_SIZE, HEAD_SIZE),
                         lambda b, t: (b, 0, 0, 0)),
        ],
    )
    return pl.pallas_call(
        _patch_body,
        grid_spec=grid_spec,
        out_shape=[
            jax.ShapeDtypeStruct(
                (NUM_BLOCKS, NUM_HEADS, BLOCK_SIZE, HEAD_SIZE), jnp.float32),
            jax.ShapeDtypeStruct(
                (NUM_BLOCKS, NUM_HEADS, BLOCK_SIZE, HEAD_SIZE), jnp.float32),
        ],
        compiler_params=pltpu.CompilerParams(
            dimension_semantics=("arbitrary",)),
    )(tok_map, *([k3] * BLOCK_SIZE), *([v3] * BLOCK_SIZE))


GEN_BLOCKS_PER_STEP = 8


def _gen_body(bt_ref, ctx_ref, *refs):
    nb = GEN_BLOCKS_PER_STEP
    q_ref = refs[0]
    kc_refs = refs[1:1 + nb]
    vc_refs = refs[1 + nb:1 + 2 * nb]
    kp_refs = refs[1 + 2 * nb:1 + 3 * nb]
    vp_refs = refs[1 + 3 * nb:1 + 4 * nb]
    pm_refs = refs[1 + 4 * nb:1 + 5 * nb]
    o_ref = refs[1 + 5 * nb]
    m_ref, l_ref, acc_ref = refs[2 + 5 * nb:]
    g = pl.program_id(0)
    j = pl.program_id(1)
    ctx = ctx_ref[g]

    @pl.when(j == 0)
    def _init():
        m_ref[...] = jnp.full_like(m_ref, NEG_INF)
        l_ref[...] = jnp.zeros_like(l_ref)
        acc_ref[...] = jnp.zeros_like(acc_ref)

    @pl.when(j * nb * BLOCK_SIZE < ctx)
    def _compute():
        q = q_ref[0] * SCALE
        qtile = jnp.broadcast_to(
            q.reshape(NUM_HEADS, HEAD_SIZE // X, 1, X),
            (NUM_HEADS, HEAD_SIZE // X, BLOCK_SIZE, X),
        ).reshape(NUM_HEADS, HEAD_SIZE // X, BLOCK_SIZE * X)
        hh = jax.lax.broadcasted_iota(
            jnp.int32, (NUM_HEADS, BLOCK_SIZE, NUM_HEADS), 0)
        hh2 = jax.lax.broadcasted_iota(
            jnp.int32, (NUM_HEADS, BLOCK_SIZE, NUM_HEADS), 2)
        eye3 = (hh == hh2).astype(jnp.float32)
        m = m_ref[:, 0:1]
        l = l_ref[:, 0:1]
        acc = acc_ref[...]
        for s in range(nb):
            kv = kc_refs[s][0]
            part = (kv * qtile).sum(axis=1)
            logits_old = part.reshape(NUM_HEADS, BLOCK_SIZE, X).sum(axis=2)
            kpmat = kp_refs[s][0].reshape(NUM_HEADS * BLOCK_SIZE, HEAD_SIZE)
            mm = jax.lax.dot_general(kpmat, q, (((1,), (1,)), ((), ())),
                                     preferred_element_type=jnp.float32)
            mm3 = mm.reshape(NUM_HEADS, BLOCK_SIZE, NUM_HEADS)
            logits_new = (mm3 * eye3).sum(axis=2)
            pmf = pm_refs[s][0]
            logits = logits_old + pmf * (logits_new - logits_old)
            tpos = (j * nb + s) * BLOCK_SIZE + jax.lax.broadcasted_iota(
                jnp.int32, (NUM_HEADS, BLOCK_SIZE), 1)
            logits = jnp.where(tpos < ctx, logits, NEG_INF)
            m_new = jnp.maximum(m, jnp.max(logits, axis=1, keepdims=True))
            p = jnp.exp(logits - m_new)
            alpha = jnp.exp(m - m_new)
            l = l * alpha + jnp.sum(p, axis=1, keepdims=True)
            p_new = p * pmf
            p_old = p - p_new
            pv = (vp_refs[s][0] * p_new[:, :, None]).sum(axis=1) + \
                 (vc_refs[s][0] * p_old[:, :, None]).sum(axis=1)
            acc = acc * alpha + pv
            m = m_new
        acc_ref[...] = acc
        m_ref[...] = jnp.broadcast_to(m, m_ref.shape)
        l_ref[...] = jnp.broadcast_to(l, l_ref.shape)

    @pl.when(j == (MAX_CTX // (BLOCK_SIZE * nb)) - 1)
    def _finish():
        o_ref[0] = acc_ref[...] / l_ref[:, 0:1]


def _gen_attention(qg, key_cache, value_cache, kpatch, vpatch, pmask,
                   block_tables, context_lens):
    nb = GEN_BLOCKS_PER_STEP
    nj = MAX_CTX // (BLOCK_SIZE * nb)

    def _jc(g, j, cl):
        return jnp.minimum(j, (cl[g] - 1) // (BLOCK_SIZE * nb))

    def _map4(s):
        return lambda g, j, bt, cl: (bt[g, _jc(g, j, cl) * nb + s], 0, 0, 0)

    def _map3(s):
        return lambda g, j, bt, cl: (bt[g, _jc(g, j, cl) * nb + s], 0, 0)

    grid_spec = pltpu.PrefetchScalarGridSpec(
        num_scalar_prefetch=2,
        grid=(NUM_GEN, nj),
        in_specs=[
            pl.BlockSpec((1, NUM_HEADS, HEAD_SIZE),
                         lambda g, j, bt, cl: (g, 0, 0)),
        ] + [
            pl.BlockSpec((1, NUM_HEADS, HEAD_SIZE // X, BLOCK_SIZE * X),
                         _map4(s)) for s in range(nb)
        ] + [
            pl.BlockSpec((1, NUM_HEADS, BLOCK_SIZE, HEAD_SIZE),
                         _map4(s)) for s in range(nb)
        ] + [
            pl.BlockSpec((1, NUM_HEADS, BLOCK_SIZE, HEAD_SIZE),
                         _map4(s)) for s in range(nb)
        ] + [
            pl.BlockSpec((1, NUM_HEADS, BLOCK_SIZE, HEAD_SIZE),
                         _map4(s)) for s in range(nb)
        ] + [
            pl.BlockSpec((1, NUM_HEADS, BLOCK_SIZE), _map3(s)) for s in range(nb)
        ],
        out_specs=pl.BlockSpec((1, NUM_HEADS, HEAD_SIZE),
                               lambda g, j, bt, cl: (g, 0, 0)),
        scratch_shapes=[
            pltpu.VMEM((NUM_HEADS, HEAD_SIZE), jnp.float32),
            pltpu.VMEM((NUM_HEADS, HEAD_SIZE), jnp.float32),
            pltpu.VMEM((NUM_HEADS, HEAD_SIZE), jnp.float32),
        ],
    )
    return pl.pallas_call(
        _gen_body,
        grid_spec=grid_spec,
        out_shape=jax.ShapeDtypeStruct((NUM_GEN, NUM_HEADS, HEAD_SIZE),
                                       jnp.float32),
        compiler_params=pltpu.CompilerParams(
            dimension_semantics=("arbitrary", "arbitrary")),
    )(block_tables.astype(jnp.int32), context_lens, qg,
      *([key_cache.reshape(NUM_BLOCKS, NUM_HEADS, HEAD_SIZE // X,
                           BLOCK_SIZE * X)] * nb),
      *([value_cache] * nb),
      *([kpatch] * nb),
      *([vpatch] * nb),
      *([pmask] * nb))


@jax.jit
def kernel(query, key, value, key_cache, value_cache, slot_mapping,
           block_tables, context_lens):
    n_tok = query.shape[0]
    start = NUM_PROMPTS * PROMPT_LEN
    q = query.reshape(n_tok, NUM_HEADS, HEAD_SIZE)
    k = key.reshape(n_tok, NUM_HEADS, HEAD_SIZE)
    v = value.reshape(n_tok, NUM_HEADS, HEAD_SIZE)

    qp = q[:start].reshape(NUM_PROMPTS, PROMPT_LEN, NUM_HEADS, HEAD_SIZE)
    kp = k[:start].reshape(NUM_PROMPTS, PROMPT_LEN, NUM_HEADS, HEAD_SIZE)
    vp = v[:start].reshape(NUM_PROMPTS, PROMPT_LEN, NUM_HEADS, HEAD_SIZE)
    qp = qp.transpose(0, 2, 1, 3)
    kp = kp.transpose(0, 2, 1, 3)
    vp = vp.transpose(0, 2, 1, 3)
    out_p = _prompt_attention(qp, kp, vp)
    out_p = out_p.transpose(0, 2, 1, 3).reshape(start, NUM_HEADS * HEAD_SIZE)

    kpatch, vpatch = _build_patches(k, v, slot_mapping)
    pmask = jnp.zeros((NUM_BLOCKS * BLOCK_SIZE,), jnp.float32)
    pmask = pmask.at[slot_mapping].set(1.0)
    pmask = jnp.broadcast_to(
        pmask.reshape(NUM_BLOCKS, 1, BLOCK_SIZE),
        (NUM_BLOCKS, NUM_HEADS, BLOCK_SIZE))

    out_g = _gen_attention(q[start:], key_cache, value_cache,
                           kpatch, vpatch, pmask, block_tables, context_lens)
    out_g = out_g.reshape(NUM_GEN, NUM_HEADS * HEAD_SIZE)

    return jnp.concatenate([out_p, out_g], axis=0)


# speedup vs baseline: 1.0342x; 1.0342x over previous
import functools

import jax
import jax.numpy as jnp
from jax.experimental import pallas as pl
from jax.experimental.pallas import tpu as pltpu

SCALE = 0.08838834764831845
NUM_HEADS = 16
HEAD_SIZE = 128
NUM_PROMPTS = 2
PROMPT_LEN = 2048
NUM_GEN = 16
BLOCK_SIZE = 16
X = 8
NUM_BLOCKS = 512
MAX_CTX = 1024
NEG_INF = -1e30

QBLK = 512
KBLK = 512


def _flash_body(q_ref, k_ref, v_ref, o_ref):
    qi = pl.program_id(2)
    q = q_ref[0, 0, :, :] * SCALE

    def body(kb, carry):
        m, l, acc = carry
        k = k_ref[0, 0, pl.ds(kb * KBLK, KBLK), :]
        v = v_ref[0, 0, pl.ds(kb * KBLK, KBLK), :]
        s = jax.lax.dot_general(q.astype(jnp.bfloat16), k.astype(jnp.bfloat16),
                                (((1,), (1,)), ((), ())),
                                preferred_element_type=jnp.float32)
        qpos = qi * QBLK + jax.lax.broadcasted_iota(jnp.int32, (QBLK, KBLK), 0)
        kpos = kb * KBLK + jax.lax.broadcasted_iota(jnp.int32, (QBLK, KBLK), 1)
        s = jnp.where(qpos >= kpos, s, NEG_INF)
        m_new = jnp.maximum(m, jnp.max(s, axis=1, keepdims=True))
        p = jnp.exp(s - m_new)
        alpha = jnp.exp(m - m_new)
        l_new = l * alpha + jnp.sum(p, axis=1, keepdims=True)
        acc_new = acc * alpha + jax.lax.dot_general(
            p.astype(jnp.bfloat16), v.astype(jnp.bfloat16),
            (((1,), (0,)), ((), ())), preferred_element_type=jnp.float32)
        return m_new, l_new, acc_new

    m0 = jnp.full((QBLK, 1), NEG_INF, jnp.float32)
    l0 = jnp.zeros((QBLK, 1), jnp.float32)
    a0 = jnp.zeros((QBLK, HEAD_SIZE), jnp.float32)
    m, l, acc = jax.lax.fori_loop(0, qi + 1, body, (m0, l0, a0))
    o_ref[0, 0, :, :] = acc / l


def _prompt_attention(qp, kp, vp):
    grid = (NUM_PROMPTS, NUM_HEADS, PROMPT_LEN // QBLK)
    return pl.pallas_call(
        _flash_body,
        grid=grid,
        in_specs=[
            pl.BlockSpec((1, 1, QBLK, HEAD_SIZE), lambda p, h, q: (p, h, q, 0)),
            pl.BlockSpec((1, 1, PROMPT_LEN, HEAD_SIZE), lambda p, h, q: (p, h, 0, 0)),
            pl.BlockSpec((1, 1, PROMPT_LEN, HEAD_SIZE), lambda p, h, q: (p, h, 0, 0)),
        ],
        out_specs=pl.BlockSpec((1, 1, QBLK, HEAD_SIZE), lambda p, h, q: (p, h, q, 0)),
        out_shape=jax.ShapeDtypeStruct(
            (NUM_PROMPTS, NUM_HEADS, PROMPT_LEN, HEAD_SIZE), jnp.float32),
        compiler_params=pltpu.CompilerParams(
            dimension_semantics=("parallel", "parallel", "arbitrary")),
    )(qp, kp, vp)


def _patch_body(tok_ref, *refs):
    kr_refs = refs[:BLOCK_SIZE]
    vr_refs = refs[BLOCK_SIZE:2 * BLOCK_SIZE]
    kp_ref, vp_ref = refs[2 * BLOCK_SIZE:]
    for off in range(BLOCK_SIZE):
        kp_ref[0, :, off, :] = kr_refs[off][0]
        vp_ref[0, :, off, :] = vr_refs[off][0]


def _build_patches(k3, v3, slot_mapping):
    tok_map = jnp.zeros((NUM_BLOCKS * BLOCK_SIZE,), jnp.int32)
    tok_map = tok_map.at[slot_mapping].set(
        jnp.arange(k3.shape[0], dtype=jnp.int32))

    def _row_map(off):
        return lambda b, t: (t[b * BLOCK_SIZE + off], 0, 0)

    grid_spec = pltpu.PrefetchScalarGridSpec(
        num_scalar_prefetch=1,
        grid=(NUM_BLOCKS,),
        in_specs=[
            pl.BlockSpec((1, NUM_HEADS, HEAD_SIZE), _row_map(off))
            for off in range(BLOCK_SIZE)
        ] + [
            pl.BlockSpec((1, NUM_HEADS, HEAD_SIZE), _row_map(off))
            for off in range(BLOCK_SIZE)
        ],
        out_specs=[
            pl.BlockSpec((1, NUM_HEADS, BLOCK_SIZE, HEAD_SIZE),
                         lambda b, t: (b, 0, 0, 0)),
            pl.BlockSpec((1, NUM_HEADS, BLOCK_SIZE, HEAD_SIZE),
                         lambda b, t: (b, 0, 0, 0)),
        ],
    )
    return pl.pallas_call(
        _patch_body,
        grid_spec=grid_spec,
        out_shape=[
            jax.ShapeDtypeStruct(
                (NUM_BLOCKS, NUM_HEADS, BLOCK_SIZE, HEAD_SIZE), jnp.float32),
            jax.ShapeDtypeStruct(
                (NUM_BLOCKS, NUM_HEADS, BLOCK_SIZE, HEAD_SIZE), jnp.float32),
        ],
        compiler_params=pltpu.CompilerParams(
            dimension_semantics=("arbitrary",)),
    )(tok_map, *([k3] * BLOCK_SIZE), *([v3] * BLOCK_SIZE))


GEN_BLOCKS_PER_STEP = 8


def _gen_body(bt_ref, ctx_ref, *refs):
    nb = GEN_BLOCKS_PER_STEP
    q_ref = refs[0]
    kc_refs = refs[1:1 + nb]
    vc_refs = refs[1 + nb:1 + 2 * nb]
    kp_refs = refs[1 + 2 * nb:1 + 3 * nb]
    vp_refs = refs[1 + 3 * nb:1 + 4 * nb]
    pm_refs = refs[1 + 4 * nb:1 + 5 * nb]
    o_ref = refs[1 + 5 * nb]
    m_ref, l_ref, acc_ref = refs[2 + 5 * nb:]
    g = pl.program_id(0)
    j = pl.program_id(1)
    ctx = ctx_ref[g]

    @pl.when(j == 0)
    def _init():
        m_ref[...] = jnp.full_like(m_ref, NEG_INF)
        l_ref[...] = jnp.zeros_like(l_ref)
        acc_ref[...] = jnp.zeros_like(acc_ref)

    @pl.when(j * nb * BLOCK_SIZE < ctx)
    def _compute():
        q = q_ref[0] * SCALE
        qtile = jnp.broadcast_to(
            q.reshape(NUM_HEADS, HEAD_SIZE // X, 1, X),
            (NUM_HEADS, HEAD_SIZE // X, BLOCK_SIZE, X),
        ).reshape(NUM_HEADS, HEAD_SIZE // X, BLOCK_SIZE * X)
        hh = jax.lax.broadcasted_iota(
            jnp.int32, (NUM_HEADS, BLOCK_SIZE, NUM_HEADS), 0)
        hh2 = jax.lax.broadcasted_iota(
            jnp.int32, (NUM_HEADS, BLOCK_SIZE, NUM_HEADS), 2)
        eye3 = (hh == hh2).astype(jnp.float32)
        m = m_ref[:, 0:1]
        l = l_ref[:, 0:1]
        acc = acc_ref[...]
        for s in range(nb):
            kv = kc_refs[s][0]
            part = (kv * qtile).sum(axis=1)
            logits_old = part.reshape(NUM_HEADS, BLOCK_SIZE, X).sum(axis=2)
            kpmat = kp_refs[s][0].reshape(NUM_HEADS * BLOCK_SIZE, HEAD_SIZE)
            mm = jax.lax.dot_general(kpmat, q, (((1,), (1,)), ((), ())),
                                     preferred_element_type=jnp.float32)
            mm3 = mm.reshape(NUM_HEADS, BLOCK_SIZE, NUM_HEADS)
            logits_new = (mm3 * eye3).sum(axis=2)
            pmf = pm_refs[s][0]
            logits = logits_old + pmf * (logits_new - logits_old)
            tpos = (j * nb + s) * BLOCK_SIZE + jax.lax.broadcasted_iota(
                jnp.int32, (NUM_HEADS, BLOCK_SIZE), 1)
            logits = jnp.where(tpos < ctx, logits, NEG_INF)
            m_new = jnp.maximum(m, jnp.max(logits, axis=1, keepdims=True))
            p = jnp.exp(logits - m_new)
            alpha = jnp.exp(m - m_new)
            l = l * alpha + jnp.sum(p, axis=1, keepdims=True)
            p_new = p * pmf
            p_old = p - p_new
            pv = (vp_refs[s][0] * p_new[:, :, None]).sum(axis=1) + \
                 (vc_refs[s][0] * p_old[:, :, None]).sum(axis=1)
            acc = acc * alpha + pv
            m = m_new
        acc_ref[...] = acc
        m_ref[...] = jnp.broadcast_to(m, m_ref.shape)
        l_ref[...] = jnp.broadcast_to(l, l_ref.shape)

    @pl.when(j == (MAX_CTX // (BLOCK_SIZE * nb)) - 1)
    def _finish():
        o_ref[0] = acc_ref[...] / l_ref[:, 0:1]


def _gen_attention(qg, key_cache, value_cache, kpatch, vpatch, pmask,
                   block_tables, context_lens):
    nb = GEN_BLOCKS_PER_STEP
    nj = MAX_CTX // (BLOCK_SIZE * nb)

    def _map4(s):
        return lambda g, j, bt, cl: (bt[g, j * nb + s], 0, 0, 0)

    def _map3(s):
        return lambda g, j, bt, cl: (bt[g, j * nb + s], 0, 0)

    grid_spec = pltpu.PrefetchScalarGridSpec(
        num_scalar_prefetch=2,
        grid=(NUM_GEN, nj),
        in_specs=[
            pl.BlockSpec((1, NUM_HEADS, HEAD_SIZE),
                         lambda g, j, bt, cl: (g, 0, 0)),
        ] + [
            pl.BlockSpec((1, NUM_HEADS, HEAD_SIZE // X, BLOCK_SIZE * X),
                         _map4(s)) for s in range(nb)
        ] + [
            pl.BlockSpec((1, NUM_HEADS, BLOCK_SIZE, HEAD_SIZE),
                         _map4(s)) for s in range(nb)
        ] + [
            pl.BlockSpec((1, NUM_HEADS, BLOCK_SIZE, HEAD_SIZE),
                         _map4(s)) for s in range(nb)
        ] + [
            pl.BlockSpec((1, NUM_HEADS, BLOCK_SIZE, HEAD_SIZE),
                         _map4(s)) for s in range(nb)
        ] + [
            pl.BlockSpec((1, NUM_HEADS, BLOCK_SIZE), _map3(s)) for s in range(nb)
        ],
        out_specs=pl.BlockSpec((1, NUM_HEADS, HEAD_SIZE),
                               lambda g, j, bt, cl: (g, 0, 0)),
        scratch_shapes=[
            pltpu.VMEM((NUM_HEADS, HEAD_SIZE), jnp.float32),
            pltpu.VMEM((NUM_HEADS, HEAD_SIZE), jnp.float32),
            pltpu.VMEM((NUM_HEADS, HEAD_SIZE), jnp.float32),
        ],
    )
    return pl.pallas_call(
        _gen_body,
        grid_spec=grid_spec,
        out_shape=jax.ShapeDtypeStruct((NUM_GEN, NUM_HEADS, HEAD_SIZE),
                                       jnp.float32),
        compiler_params=pltpu.CompilerParams(
            dimension_semantics=("arbitrary", "arbitrary")),
    )(block_tables.astype(jnp.int32), context_lens, qg,
      *([key_cache.reshape(NUM_BLOCKS, NUM_HEADS, HEAD_SIZE // X,
                           BLOCK_SIZE * X)] * nb),
      *([value_cache] * nb),
      *([kpatch] * nb),
      *([vpatch] * nb),
      *([pmask] * nb))


@jax.jit
def kernel(query, key, value, key_cache, value_cache, slot_mapping,
           block_tables, context_lens):
    n_tok = query.shape[0]
    start = NUM_PROMPTS * PROMPT_LEN
    q = query.reshape(n_tok, NUM_HEADS, HEAD_SIZE)
    k = key.reshape(n_tok, NUM_HEADS, HEAD_SIZE)
    v = value.reshape(n_tok, NUM_HEADS, HEAD_SIZE)

    qp = q[:start].reshape(NUM_PROMPTS, PROMPT_LEN, NUM_HEADS, HEAD_SIZE)
    kp = k[:start].reshape(NUM_PROMPTS, PROMPT_LEN, NUM_HEADS, HEAD_SIZE)
    vp = v[:start].reshape(NUM_PROMPTS, PROMPT_LEN, NUM_HEADS, HEAD_SIZE)
    qp = qp.transpose(0, 2, 1, 3)
    kp = kp.transpose(0, 2, 1, 3)
    vp = vp.transpose(0, 2, 1, 3)
    out_p = _prompt_attention(qp, kp, vp)
    out_p = out_p.transpose(0, 2, 1, 3).reshape(start, NUM_HEADS * HEAD_SIZE)

    kpatch, vpatch = _build_patches(k, v, slot_mapping)
    pmask = jnp.zeros((NUM_BLOCKS * BLOCK_SIZE,), jnp.float32)
    pmask = pmask.at[slot_mapping].set(1.0)
    pmask = jnp.broadcast_to(
        pmask.reshape(NUM_BLOCKS, 1, BLOCK_SIZE),
        (NUM_BLOCKS, NUM_HEADS, BLOCK_SIZE))

    out_g = _gen_attention(q[start:], key_cache, value_cache,
                           kpatch, vpatch, pmask, block_tables, context_lens)
    out_g = out_g.reshape(NUM_GEN, NUM_HEADS * HEAD_SIZE)

    return jnp.concatenate([out_p, out_g], axis=0)


# speedup vs baseline: 1.0484x; 1.0137x over previous
import functools

import jax
import jax.numpy as jnp
from jax.experimental import pallas as pl
from jax.experimental.pallas import tpu as pltpu

SCALE = 0.08838834764831845
NUM_HEADS = 16
HEAD_SIZE = 128
NUM_PROMPTS = 2
PROMPT_LEN = 2048
NUM_GEN = 16
BLOCK_SIZE = 16
X = 8
NUM_BLOCKS = 512
MAX_CTX = 1024
NEG_INF = -1e30

QBLK = 512
KBLK = 512


def _flash_body(q_ref, k_ref, v_ref, o_ref):
    qi = pl.program_id(2)
    q = q_ref[0, 0, :, :] * SCALE

    def body(kb, carry):
        m, l, acc = carry
        k = k_ref[0, 0, pl.ds(kb * KBLK, KBLK), :]
        v = v_ref[0, 0, pl.ds(kb * KBLK, KBLK), :]
        s = jax.lax.dot_general(q, k, (((1,), (1,)), ((), ())),
                                preferred_element_type=jnp.float32)
        qpos = qi * QBLK + jax.lax.broadcasted_iota(jnp.int32, (QBLK, KBLK), 0)
        kpos = kb * KBLK + jax.lax.broadcasted_iota(jnp.int32, (QBLK, KBLK), 1)
        s = jnp.where(qpos >= kpos, s, NEG_INF)
        m_new = jnp.maximum(m, jnp.max(s, axis=1, keepdims=True))
        p = jnp.exp(s - m_new)
        alpha = jnp.exp(m - m_new)
        l_new = l * alpha + jnp.sum(p, axis=1, keepdims=True)
        acc_new = acc * alpha + jax.lax.dot_general(
            p, v, (((1,), (0,)), ((), ())), preferred_element_type=jnp.float32)
        return m_new, l_new, acc_new

    m0 = jnp.full((QBLK, 1), NEG_INF, jnp.float32)
    l0 = jnp.zeros((QBLK, 1), jnp.float32)
    a0 = jnp.zeros((QBLK, HEAD_SIZE), jnp.float32)
    m, l, acc = jax.lax.fori_loop(0, qi + 1, body, (m0, l0, a0))
    o_ref[0, 0, :, :] = acc / l


def _prompt_attention(qp, kp, vp):
    grid = (NUM_PROMPTS, NUM_HEADS, PROMPT_LEN // QBLK)
    return pl.pallas_call(
        _flash_body,
        grid=grid,
        in_specs=[
            pl.BlockSpec((1, 1, QBLK, HEAD_SIZE), lambda p, h, q: (p, h, q, 0)),
            pl.BlockSpec((1, 1, PROMPT_LEN, HEAD_SIZE), lambda p, h, q: (p, h, 0, 0)),
            pl.BlockSpec((1, 1, PROMPT_LEN, HEAD_SIZE), lambda p, h, q: (p, h, 0, 0)),
        ],
        out_specs=pl.BlockSpec((1, 1, QBLK, HEAD_SIZE), lambda p, h, q: (p, h, q, 0)),
        out_shape=jax.ShapeDtypeStruct(
            (NUM_PROMPTS, NUM_HEADS, PROMPT_LEN, HEAD_SIZE), jnp.float32),
        compiler_params=pltpu.CompilerParams(
            dimension_semantics=("parallel", "parallel", "arbitrary")),
    )(qp, kp, vp)


def _patch_body(tok_ref, pm_ref, *refs):
    kr_refs = refs[:BLOCK_SIZE]
    vr_refs = refs[BLOCK_SIZE:2 * BLOCK_SIZE]
    vc_in_ref = refs[2 * BLOCK_SIZE]
    kp_ref, vc2_ref = refs[2 * BLOCK_SIZE + 1:]
    b = pl.program_id(0)
    vc2_ref[...] = vc_in_ref[...]
    for off in range(BLOCK_SIZE):
        kp_ref[0, :, off, :] = kr_refs[off][0]

        @pl.when(pm_ref[b * BLOCK_SIZE + off] == 1)
        def _patch_v():
            vc2_ref[0, :, off, :] = vr_refs[off][0]


def _build_patches(k3, v3, value_cache, slot_mapping):
    tok_map = jnp.zeros((NUM_BLOCKS * BLOCK_SIZE,), jnp.int32)
    tok_map = tok_map.at[slot_mapping].set(
        jnp.arange(k3.shape[0], dtype=jnp.int32))
    pm = jnp.zeros((NUM_BLOCKS * BLOCK_SIZE,), jnp.int32)
    pm = pm.at[slot_mapping].set(1)

    def _row_map(off):
        return lambda b, t, m: (t[b * BLOCK_SIZE + off], 0, 0)

    grid_spec = pltpu.PrefetchScalarGridSpec(
        num_scalar_prefetch=2,
        grid=(NUM_BLOCKS,),
        in_specs=[
            pl.BlockSpec((1, NUM_HEADS, HEAD_SIZE), _row_map(off))
            for off in range(BLOCK_SIZE)
        ] + [
            pl.BlockSpec((1, NUM_HEADS, HEAD_SIZE), _row_map(off))
            for off in range(BLOCK_SIZE)
        ] + [
            pl.BlockSpec((1, NUM_HEADS, BLOCK_SIZE, HEAD_SIZE),
                         lambda b, t, m: (b, 0, 0, 0)),
        ],
        out_specs=[
            pl.BlockSpec((1, NUM_HEADS, BLOCK_SIZE, HEAD_SIZE),
                         lambda b, t, m: (b, 0, 0, 0)),
            pl.BlockSpec((1, NUM_HEADS, BLOCK_SIZE, HEAD_SIZE),
                         lambda b, t, m: (b, 0, 0, 0)),
        ],
    )
    return pl.pallas_call(
        _patch_body,
        grid_spec=grid_spec,
        out_shape=[
            jax.ShapeDtypeStruct(
                (NUM_BLOCKS, NUM_HEADS, BLOCK_SIZE, HEAD_SIZE), jnp.float32),
            jax.ShapeDtypeStruct(
                (NUM_BLOCKS, NUM_HEADS, BLOCK_SIZE, HEAD_SIZE), jnp.float32),
        ],
        compiler_params=pltpu.CompilerParams(
            dimension_semantics=("arbitrary",)),
    )(tok_map, pm, *([k3] * BLOCK_SIZE), *([v3] * BLOCK_SIZE), value_cache)


GEN_BLOCKS_PER_STEP = 8


def _gen_body(bt_ref, ctx_ref, *refs):
    nb = GEN_BLOCKS_PER_STEP
    q_ref = refs[0]
    kc_refs = refs[1:1 + nb]
    vc_refs = refs[1 + nb:1 + 2 * nb]
    kp_refs = refs[1 + 2 * nb:1 + 3 * nb]
    pm_refs = refs[1 + 3 * nb:1 + 4 * nb]
    o_ref = refs[1 + 4 * nb]
    m_ref, l_ref, acc_ref = refs[2 + 4 * nb:]
    g = pl.program_id(0)
    j = pl.program_id(1)
    ctx = ctx_ref[g]

    @pl.when(j == 0)
    def _init():
        m_ref[...] = jnp.full_like(m_ref, NEG_INF)
        l_ref[...] = jnp.zeros_like(l_ref)
        acc_ref[...] = jnp.zeros_like(acc_ref)

    @pl.when(j * nb * BLOCK_SIZE < ctx)
    def _compute():
        q = q_ref[0] * SCALE
        qtile = jnp.broadcast_to(
            q.reshape(NUM_HEADS, HEAD_SIZE // X, 1, X),
            (NUM_HEADS, HEAD_SIZE // X, BLOCK_SIZE, X),
        ).reshape(NUM_HEADS, HEAD_SIZE // X, BLOCK_SIZE * X)
        hh = jax.lax.broadcasted_iota(
            jnp.int32, (NUM_HEADS, BLOCK_SIZE, NUM_HEADS), 0)
        hh2 = jax.lax.broadcasted_iota(
            jnp.int32, (NUM_HEADS, BLOCK_SIZE, NUM_HEADS), 2)
        eye3 = (hh == hh2).astype(jnp.float32)
        m = m_ref[:, 0:1]
        l = l_ref[:, 0:1]
        acc = acc_ref[...]
        for s in range(nb):
            kv = kc_refs[s][0]
            part = (kv * qtile).sum(axis=1)
            logits_old = part.reshape(NUM_HEADS, BLOCK_SIZE, X).sum(axis=2)
            kpmat = kp_refs[s][0].reshape(NUM_HEADS * BLOCK_SIZE, HEAD_SIZE)
            mm = jax.lax.dot_general(kpmat, q, (((1,), (1,)), ((), ())),
                                     preferred_element_type=jnp.float32)
            mm3 = mm.reshape(NUM_HEADS, BLOCK_SIZE, NUM_HEADS)
            logits_new = (mm3 * eye3).sum(axis=2)
            pmf = pm_refs[s][0]
            logits = logits_old + pmf * (logits_new - logits_old)
            tpos = (j * nb + s) * BLOCK_SIZE + jax.lax.broadcasted_iota(
                jnp.int32, (NUM_HEADS, BLOCK_SIZE), 1)
            logits = jnp.where(tpos < ctx, logits, NEG_INF)
            m_new = jnp.maximum(m, jnp.max(logits, axis=1, keepdims=True))
            p = jnp.exp(logits - m_new)
            alpha = jnp.exp(m - m_new)
            l = l * alpha + jnp.sum(p, axis=1, keepdims=True)
            pv = (vc_refs[s][0] * p[:, :, None]).sum(axis=1)
            acc = acc * alpha + pv
            m = m_new
        acc_ref[...] = acc
        m_ref[...] = jnp.broadcast_to(m, m_ref.shape)
        l_ref[...] = jnp.broadcast_to(l, l_ref.shape)

    @pl.when(j == (MAX_CTX // (BLOCK_SIZE * nb)) - 1)
    def _finish():
        o_ref[0] = acc_ref[...] / l_ref[:, 0:1]


def _gen_attention(qg, key_cache, value_cache, kpatch, pmask,
                   block_tables, context_lens):
    nb = GEN_BLOCKS_PER_STEP
    nj = MAX_CTX // (BLOCK_SIZE * nb)

    def _map4(s):
        return lambda g, j, bt, cl: (bt[g, j * nb + s], 0, 0, 0)

    def _map3(s):
        return lambda g, j, bt, cl: (bt[g, j * nb + s], 0, 0)

    grid_spec = pltpu.PrefetchScalarGridSpec(
        num_scalar_prefetch=2,
        grid=(NUM_GEN, nj),
        in_specs=[
            pl.BlockSpec((1, NUM_HEADS, HEAD_SIZE),
                         lambda g, j, bt, cl: (g, 0, 0)),
        ] + [
            pl.BlockSpec((1, NUM_HEADS, HEAD_SIZE // X, BLOCK_SIZE * X),
                         _map4(s)) for s in range(nb)
        ] + [
            pl.BlockSpec((1, NUM_HEADS, BLOCK_SIZE, HEAD_SIZE),
                         _map4(s)) for s in range(nb)
        ] + [
            pl.BlockSpec((1, NUM_HEADS, BLOCK_SIZE, HEAD_SIZE),
                         _map4(s)) for s in range(nb)
        ] + [
            pl.BlockSpec((1, NUM_HEADS, BLOCK_SIZE), _map3(s)) for s in range(nb)
        ],
        out_specs=pl.BlockSpec((1, NUM_HEADS, HEAD_SIZE),
                               lambda g, j, bt, cl: (g, 0, 0)),
        scratch_shapes=[
            pltpu.VMEM((NUM_HEADS, HEAD_SIZE), jnp.float32),
            pltpu.VMEM((NUM_HEADS, HEAD_SIZE), jnp.float32),
            pltpu.VMEM((NUM_HEADS, HEAD_SIZE), jnp.float32),
        ],
    )
    return pl.pallas_call(
        _gen_body,
        grid_spec=grid_spec,
        out_shape=jax.ShapeDtypeStruct((NUM_GEN, NUM_HEADS, HEAD_SIZE),
                                       jnp.float32),
        compiler_params=pltpu.CompilerParams(
            dimension_semantics=("arbitrary", "arbitrary")),
    )(block_tables.astype(jnp.int32), context_lens, qg,
      *([key_cache.reshape(NUM_BLOCKS, NUM_HEADS, HEAD_SIZE // X,
                           BLOCK_SIZE * X)] * nb),
      *([value_cache] * nb),
      *([kpatch] * nb),
      *([pmask] * nb))


@jax.jit
def kernel(query, key, value, key_cache, value_cache, slot_mapping,
           block_tables, context_lens):
    n_tok = query.shape[0]
    start = NUM_PROMPTS * PROMPT_LEN
    q = query.reshape(n_tok, NUM_HEADS, HEAD_SIZE)
    k = key.reshape(n_tok, NUM_HEADS, HEAD_SIZE)
    v = value.reshape(n_tok, NUM_HEADS, HEAD_SIZE)

    qp = q[:start].reshape(NUM_PROMPTS, PROMPT_LEN, NUM_HEADS, HEAD_SIZE)
    kp = k[:start].reshape(NUM_PROMPTS, PROMPT_LEN, NUM_HEADS, HEAD_SIZE)
    vp = v[:start].reshape(NUM_PROMPTS, PROMPT_LEN, NUM_HEADS, HEAD_SIZE)
    qp = qp.transpose(0, 2, 1, 3)
    kp = kp.transpose(0, 2, 1, 3)
    vp = vp.transpose(0, 2, 1, 3)
    out_p = _prompt_attention(qp, kp, vp)
    out_p = out_p.transpose(0, 2, 1, 3).reshape(start, NUM_HEADS * HEAD_SIZE)

    kpatch, vc2 = _build_patches(k, v, value_cache, slot_mapping)
    pmask = jnp.zeros((NUM_BLOCKS * BLOCK_SIZE,), jnp.float32)
    pmask = pmask.at[slot_mapping].set(1.0)
    pmask = jnp.broadcast_to(
        pmask.reshape(NUM_BLOCKS, 1, BLOCK_SIZE),
        (NUM_BLOCKS, NUM_HEADS, BLOCK_SIZE))

    out_g = _gen_attention(q[start:], key_cache, vc2,
                           kpatch, pmask, block_tables, context_lens)
    out_g = out_g.reshape(NUM_GEN, NUM_HEADS * HEAD_SIZE)

    return jnp.concatenate([out_p, out_g], axis=0)


# speedup vs baseline: 1.0489x; 1.0005x over previous
import jax
import jax.numpy as jnp
from jax.experimental import pallas as pl
from jax.experimental.pallas import tpu as pltpu

SCALE = 0.08838834764831845
NUM_HEADS = 16
HEAD_SIZE = 128
NUM_PROMPTS = 2
PROMPT_LEN = 2048
NUM_GEN = 16
BLOCK_SIZE = 16
X = 8
NUM_BLOCKS = 512
MAX_CTX = 1024
NEG_INF = -1e30

QBLK = 512
KBLK = 512


def _flash_body(q_ref, k_ref, v_ref, o_ref):
    qi = pl.program_id(2)
    q = q_ref[0, 0, :, :] * SCALE

    def body(kb, carry):
        m, l, acc = carry
        k = k_ref[0, 0, pl.ds(kb * KBLK, KBLK), :]
        v = v_ref[0, 0, pl.ds(kb * KBLK, KBLK), :]
        s = jax.lax.dot_general(q, k, (((1,), (1,)), ((), ())),
                                preferred_element_type=jnp.float32)
        qpos = qi * QBLK + jax.lax.broadcasted_iota(jnp.int32, (QBLK, KBLK), 0)
        kpos = kb * KBLK + jax.lax.broadcasted_iota(jnp.int32, (QBLK, KBLK), 1)
        s = jnp.where(qpos >= kpos, s, NEG_INF)
        m_new = jnp.maximum(m, jnp.max(s, axis=1, keepdims=True))
        p = jnp.exp(s - m_new)
        alpha = jnp.exp(m - m_new)
        l_new = l * alpha + jnp.sum(p, axis=1, keepdims=True)
        acc_new = acc * alpha + jax.lax.dot_general(
            p, v, (((1,), (0,)), ((), ())), preferred_element_type=jnp.float32)
        return m_new, l_new, acc_new

    m0 = jnp.full((QBLK, 1), NEG_INF, jnp.float32)
    l0 = jnp.zeros((QBLK, 1), jnp.float32)
    a0 = jnp.zeros((QBLK, HEAD_SIZE), jnp.float32)
    m, l, acc = jax.lax.fori_loop(0, qi + 1, body, (m0, l0, a0))
    o_ref[0, 0, :, :] = acc / l


def _prompt_attention(qp, kp, vp):
    grid = (NUM_PROMPTS, NUM_HEADS, PROMPT_LEN // QBLK)
    return pl.pallas_call(
        _flash_body,
        grid=grid,
        in_specs=[
            pl.BlockSpec((1, 1, QBLK, HEAD_SIZE), lambda p, h, q: (p, h, q, 0)),
            pl.BlockSpec((1, 1, PROMPT_LEN, HEAD_SIZE), lambda p, h, q: (p, h, 0, 0)),
            pl.BlockSpec((1, 1, PROMPT_LEN, HEAD_SIZE), lambda p, h, q: (p, h, 0, 0)),
        ],
        out_specs=pl.BlockSpec((1, 1, QBLK, HEAD_SIZE), lambda p, h, q: (p, h, q, 0)),
        out_shape=jax.ShapeDtypeStruct(
            (NUM_PROMPTS, NUM_HEADS, PROMPT_LEN, HEAD_SIZE), jnp.float32),
        compiler_params=pltpu.CompilerParams(
            dimension_semantics=("parallel", "parallel", "arbitrary")),
    )(qp, kp, vp)


def _patch_body(tok_ref, pm_ref, *refs):
    kr_refs = refs[:BLOCK_SIZE]
    vr_refs = refs[BLOCK_SIZE:2 * BLOCK_SIZE]
    vc_in_ref = refs[2 * BLOCK_SIZE]
    kp_ref, vc2_ref = refs[2 * BLOCK_SIZE + 1:]
    b = pl.program_id(0)
    vc2_ref[...] = vc_in_ref[...]
    for off in range(BLOCK_SIZE):
        kp_ref[0, :, off, :] = kr_refs[off][0]

        @pl.when(pm_ref[b * BLOCK_SIZE + off] == 1)
        def _patch_v():
            vc2_ref[0, :, off, :] = vr_refs[off][0]


def _build_patches(k3, v3, value_cache, slot_mapping):
    tok_map = jnp.zeros((NUM_BLOCKS * BLOCK_SIZE,), jnp.int32)
    tok_map = tok_map.at[slot_mapping].set(
        jnp.arange(k3.shape[0], dtype=jnp.int32))
    pm = jnp.zeros((NUM_BLOCKS * BLOCK_SIZE,), jnp.int32)
    pm = pm.at[slot_mapping].set(1)

    def _row_map(off):
        return lambda b, t, m: (t[b * BLOCK_SIZE + off], 0, 0)

    grid_spec = pltpu.PrefetchScalarGridSpec(
        num_scalar_prefetch=2,
        grid=(NUM_BLOCKS,),
        in_specs=[
            pl.BlockSpec((1, NUM_HEADS, HEAD_SIZE), _row_map(off))
            for off in range(BLOCK_SIZE)
        ] + [
            pl.BlockSpec((1, NUM_HEADS, HEAD_SIZE), _row_map(off))
            for off in range(BLOCK_SIZE)
        ] + [
            pl.BlockSpec((1, NUM_HEADS, BLOCK_SIZE, HEAD_SIZE),
                         lambda b, t, m: (b, 0, 0, 0)),
        ],
        out_specs=[
            pl.BlockSpec((1, NUM_HEADS, BLOCK_SIZE, HEAD_SIZE),
                         lambda b, t, m: (b, 0, 0, 0)),
            pl.BlockSpec((1, NUM_HEADS, BLOCK_SIZE, HEAD_SIZE),
                         lambda b, t, m: (b, 0, 0, 0)),
        ],
    )
    return pl.pallas_call(
        _patch_body,
        grid_spec=grid_spec,
        out_shape=[
            jax.ShapeDtypeStruct(
                (NUM_BLOCKS, NUM_HEADS, BLOCK_SIZE, HEAD_SIZE), jnp.float32),
            jax.ShapeDtypeStruct(
                (NUM_BLOCKS, NUM_HEADS, BLOCK_SIZE, HEAD_SIZE), jnp.float32),
        ],
        compiler_params=pltpu.CompilerParams(
            dimension_semantics=("arbitrary",)),
    )(tok_map, pm, *([k3] * BLOCK_SIZE), *([v3] * BLOCK_SIZE), value_cache)


GEN_BLOCKS_PER_STEP = 8


def _gen_body(bt_ref, ctx_ref, *refs):
    nb = GEN_BLOCKS_PER_STEP
    q_ref = refs[0]
    kc_refs = refs[1:1 + nb]
    vc_refs = refs[1 + nb:1 + 2 * nb]
    kp_refs = refs[1 + 2 * nb:1 + 3 * nb]
    pm_refs = refs[1 + 3 * nb:1 + 4 * nb]
    o_ref = refs[1 + 4 * nb]
    m_ref, l_ref, acc_ref = refs[2 + 4 * nb:]
    g = pl.program_id(0)
    j = pl.program_id(1)
    ctx = ctx_ref[g]

    @pl.when(j == 0)
    def _init():
        m_ref[...] = jnp.full_like(m_ref, NEG_INF)
        l_ref[...] = jnp.zeros_like(l_ref)
        acc_ref[...] = jnp.zeros_like(acc_ref)

    @pl.when(j * nb * BLOCK_SIZE < ctx)
    def _compute():
        q = q_ref[0] * SCALE
        qtile = jnp.broadcast_to(
            q.reshape(NUM_HEADS, HEAD_SIZE // X, 1, X),
            (NUM_HEADS, HEAD_SIZE // X, BLOCK_SIZE, X),
        ).reshape(NUM_HEADS, HEAD_SIZE // X, BLOCK_SIZE * X)
        hh = jax.lax.broadcasted_iota(
            jnp.int32, (NUM_HEADS, BLOCK_SIZE, NUM_HEADS), 0)
        hh2 = jax.lax.broadcasted_iota(
            jnp.int32, (NUM_HEADS, BLOCK_SIZE, NUM_HEADS), 2)
        eye3 = (hh == hh2).astype(jnp.float32)
        m = m_ref[:, 0:1]
        l = l_ref[:, 0:1]
        acc = acc_ref[...]
        for s in range(nb):
            kv = kc_refs[s][0]
            part = (kv * qtile).sum(axis=1)
            logits_old = part.reshape(NUM_HEADS, BLOCK_SIZE, X).sum(axis=2)
            kpmat = kp_refs[s][0].reshape(NUM_HEADS * BLOCK_SIZE, HEAD_SIZE)
            mm = jax.lax.dot_general(kpmat, q, (((1,), (1,)), ((), ())),
                                     preferred_element_type=jnp.float32)
            mm3 = mm.reshape(NUM_HEADS, BLOCK_SIZE, NUM_HEADS)
            logits_new = (mm3 * eye3).sum(axis=2)
            pmf = pm_refs[s][0]
            logits = logits_old + pmf * (logits_new - logits_old)
            tpos = (j * nb + s) * BLOCK_SIZE + jax.lax.broadcasted_iota(
                jnp.int32, (NUM_HEADS, BLOCK_SIZE), 1)
            logits = jnp.where(tpos < ctx, logits, NEG_INF)
            m_new = jnp.maximum(m, jnp.max(logits, axis=1, keepdims=True))
            p = jnp.exp(logits - m_new)
            alpha = jnp.exp(m - m_new)
            l = l * alpha + jnp.sum(p, axis=1, keepdims=True)
            pv = (vc_refs[s][0] * p[:, :, None]).sum(axis=1)
            acc = acc * alpha + pv
            m = m_new
        acc_ref[...] = acc
        m_ref[...] = jnp.broadcast_to(m, m_ref.shape)
        l_ref[...] = jnp.broadcast_to(l, l_ref.shape)

    @pl.when(j == (MAX_CTX // (BLOCK_SIZE * nb)) - 1)
    def _finish():
        o_ref[0] = acc_ref[...] / l_ref[:, 0:1]


def _gen_attention(qg, key_cache, value_cache, kpatch, pmask,
                   block_tables, context_lens):
    nb = GEN_BLOCKS_PER_STEP
    nj = MAX_CTX // (BLOCK_SIZE * nb)

    def _map4(s):
        return lambda g, j, bt, cl: (bt[g, j * nb + s], 0, 0, 0)

    def _map3(s):
        return lambda g, j, bt, cl: (bt[g, j * nb + s], 0, 0)

    grid_spec = pltpu.PrefetchScalarGridSpec(
        num_scalar_prefetch=2,
        grid=(NUM_GEN, nj),
        in_specs=[
            pl.BlockSpec((1, NUM_HEADS, HEAD_SIZE),
                         lambda g, j, bt, cl: (g, 0, 0)),
        ] + [
            pl.BlockSpec((1, NUM_HEADS, HEAD_SIZE // X, BLOCK_SIZE * X),
                         _map4(s)) for s in range(nb)
        ] + [
            pl.BlockSpec((1, NUM_HEADS, BLOCK_SIZE, HEAD_SIZE),
                         _map4(s)) for s in range(nb)
        ] + [
            pl.BlockSpec((1, NUM_HEADS, BLOCK_SIZE, HEAD_SIZE),
                         _map4(s)) for s in range(nb)
        ] + [
            pl.BlockSpec((1, NUM_HEADS, BLOCK_SIZE), _map3(s)) for s in range(nb)
        ],
        out_specs=pl.BlockSpec((1, NUM_HEADS, HEAD_SIZE),
                               lambda g, j, bt, cl: (g, 0, 0)),
        scratch_shapes=[
            pltpu.VMEM((NUM_HEADS, HEAD_SIZE), jnp.float32),
            pltpu.VMEM((NUM_HEADS, HEAD_SIZE), jnp.float32),
            pltpu.VMEM((NUM_HEADS, HEAD_SIZE), jnp.float32),
        ],
    )
    return pl.pallas_call(
        _gen_body,
        grid_spec=grid_spec,
        out_shape=jax.ShapeDtypeStruct((NUM_GEN, NUM_HEADS, HEAD_SIZE),
                                       jnp.float32),
        compiler_params=pltpu.CompilerParams(
            dimension_semantics=("arbitrary", "arbitrary")),
    )(block_tables.astype(jnp.int32), context_lens, qg,
      *([key_cache.reshape(NUM_BLOCKS, NUM_HEADS, HEAD_SIZE // X,
                           BLOCK_SIZE * X)] * nb),
      *([value_cache] * nb),
      *([kpatch] * nb),
      *([pmask] * nb))


@jax.jit
def kernel(query, key, value, key_cache, value_cache, slot_mapping,
           block_tables, context_lens):
    n_tok = query.shape[0]
    start = NUM_PROMPTS * PROMPT_LEN
    q = query.reshape(n_tok, NUM_HEADS, HEAD_SIZE)
    k = key.reshape(n_tok, NUM_HEADS, HEAD_SIZE)
    v = value.reshape(n_tok, NUM_HEADS, HEAD_SIZE)

    qp = q[:start].reshape(NUM_PROMPTS, PROMPT_LEN, NUM_HEADS, HEAD_SIZE)
    kp = k[:start].reshape(NUM_PROMPTS, PROMPT_LEN, NUM_HEADS, HEAD_SIZE)
    vp = v[:start].reshape(NUM_PROMPTS, PROMPT_LEN, NUM_HEADS, HEAD_SIZE)
    qp = qp.transpose(0, 2, 1, 3)
    kp = kp.transpose(0, 2, 1, 3)
    vp = vp.transpose(0, 2, 1, 3)
    out_p = _prompt_attention(qp, kp, vp)
    out_p = out_p.transpose(0, 2, 1, 3).reshape(start, NUM_HEADS * HEAD_SIZE)

    kpatch, vc2 = _build_patches(k, v, value_cache, slot_mapping)
    pmask = jnp.zeros((NUM_BLOCKS * BLOCK_SIZE,), jnp.float32)
    pmask = pmask.at[slot_mapping].set(1.0)
    pmask = jnp.broadcast_to(
        pmask.reshape(NUM_BLOCKS, 1, BLOCK_SIZE),
        (NUM_BLOCKS, NUM_HEADS, BLOCK_SIZE))

    out_g = _gen_attention(q[start:], key_cache, vc2,
                           kpatch, pmask, block_tables, context_lens)
    out_g = out_g.reshape(NUM_GEN, NUM_HEADS * HEAD_SIZE)

    return jnp.concatenate([out_p, out_g], axis=0)
